# Initial kernel scaffold; baseline (speedup 1.0000x reference)
import jax
import jax.numpy as jnp
from jax.experimental import pallas as pl


def kernel(x, edge_index, edge_attr, W_in, b_in, W_rel, b_rel, W_root, gamma, beta):
    raise NotImplementedError("write your pallas kernel here")


# speedup vs baseline: 1.7490x; 1.7490x over previous
import functools

import jax
import jax.numpy as jnp
from jax import lax
from jax.experimental import pallas as pl
from jax.experimental.pallas import tpu as pltpu
from jax.experimental.pallas import tpu_sc as plsc

NC = 2
NS = 16
LN = 16

_MESH = dict(core_axis_name="c", subcore_axis_name="s", num_cores=NC,
             num_subcores=NS)


def _round_up(a, b):
    return (a + b - 1) // b * b


def _make_deg(EPAD, NACC):
    ER = EPAD // 128
    ERW = ER // (NC * NS)
    SLC = NACC // NS

    def body(dstp, wp, out, acc, dv, wv, zb):
        c = lax.axis_index("c")
        s = lax.axis_index("s")
        row0 = (c * NS + s) * ERW
        pltpu.sync_copy(dstp.at[pl.ds(row0, ERW)], dv)
        pltpu.sync_copy(wp.at[pl.ds(row0, ERW)], wv)
        z = jnp.zeros((LN,), jnp.float32)

        def zloop(i, _):
            zb[pl.ds(i * LN, LN)] = z
            return 0

        lax.fori_loop(0, SLC // LN, zloop, 0)
        pltpu.sync_copy(zb, acc.at[pl.ds(s * SLC, SLC)])
        plsc.subcore_barrier()

        def jloop(j, _):
            pltpu.sync_copy(wv.at[j], acc.at[dv.at[j]], add=True)
            return 0

        lax.fori_loop(0, ERW, jloop, 0)
        plsc.subcore_barrier()
        pltpu.sync_copy(acc.at[pl.ds(s * SLC, SLC)],
                        out.at[c, pl.ds(s * SLC, SLC)])

    return pl.kernel(
        body,
        out_type=jax.ShapeDtypeStruct((NC, NACC), jnp.float32),
        mesh=plsc.VectorSubcoreMesh(**_MESH),
        scratch_types=[
            pltpu.VMEM_SHARED((NACC,), jnp.float32),
            pltpu.VMEM((ERW, 128), jnp.int32),
            pltpu.VMEM((ERW, 128), jnp.float32),
            pltpu.VMEM((SLC,), jnp.float32),
        ],
    )


def _make_norm(EPAD, NACC):
    ER = EPAD // 128
    ERW = ER // (NC * NS)
    SR = NACC // 128

    def body(srcp, dstp, wp, s_hbm, out, sv, dv, wv, ov, ssb, sdb, sem):
        c = lax.axis_index("c")
        s = lax.axis_index("s")
        row0 = (c * NS + s) * ERW
        pltpu.sync_copy(srcp.at[pl.ds(row0, ERW)], sv)
        pltpu.sync_copy(dstp.at[pl.ds(row0, ERW)], dv)
        pltpu.sync_copy(wp.at[pl.ds(row0, ERW)], wv)

        def rloop(r, _):
            d1 = pltpu.async_copy(s_hbm.at[sv.at[r]], ssb, sem)
            d2 = pltpu.async_copy(s_hbm.at[dv.at[r]], sdb, sem)
            d1.wait()
            d2.wait()
            for k in range(128 // LN):
                sl = pl.ds(k * LN, LN)
                ov[r, sl] = wv[r, sl] * ssb[sl] * sdb[sl]
            return 0

        lax.fori_loop(0, ERW, rloop, 0)
        pltpu.sync_copy(ov, out.at[pl.ds(row0, ERW)])

    return pl.kernel(
        body,
        out_type=jax.ShapeDtypeStruct((ER, 128), jnp.float32),
        mesh=plsc.VectorSubcoreMesh(**_MESH),
        scratch_types=[
            pltpu.VMEM((ERW, 128), jnp.int32),
            pltpu.VMEM((ERW, 128), jnp.int32),
            pltpu.VMEM((ERW, 128), jnp.float32),
            pltpu.VMEM((ERW, 128), jnp.float32),
            pltpu.VMEM((128,), jnp.float32),
            pltpu.VMEM((128,), jnp.float32),
            pltpu.SemaphoreType.DMA,
        ],
    )


def _make_agg(EPAD, NACC, N, HQ):
    ER = EPAD // 128
    ERT = ER // NS
    SLR = NACC // NS // 128
    HH = HQ // 2

    def body(htab, srcp, dstp, normp, out, acc, sv, dv, nv, gidx, rows, zb,
             sem):
        c = lax.axis_index("c")
        s = lax.axis_index("s")
        row0 = s * ERT
        pltpu.sync_copy(srcp.at[pl.ds(row0, ERT)], sv)
        pltpu.sync_copy(dstp.at[pl.ds(row0, ERT)], dv)
        pltpu.sync_copy(normp.at[pl.ds(row0, ERT)], nv)
        z = jnp.zeros((LN,), jnp.float32)

        def zloop(i, _):
            for k in range(HH // LN):
                zb[i, pl.ds(k * LN, LN)] = z
            return 0

        lax.fori_loop(0, 128, zloop, 0)

        for qi in range(4 // NC):
            q = c * (4 // NC) + qi
            for half in range(2):
                tb = q * 2 * N + half
                for k in range(SLR):
                    pltpu.sync_copy(
                        zb, acc.at[pl.ds((s * SLR + k) * 128, 128)])
                plsc.subcore_barrier()

                def bloop(b, _):
                    for k in range(128 // LN):
                        sl = pl.ds(k * LN, LN)
                        gidx[sl] = sv[b, sl] * 2 + tb
                    pltpu.async_copy(htab.at[gidx], rows, sem).wait()

                    def scale(k8, _):
                        n16 = nv[b, pl.ds(k8 * LN, LN)]
                        for j in range(LN):
                            nsp = lax.gather(
                                n16, jnp.full((LN, 1), j, jnp.int32),
                                lax.GatherDimensionNumbers(
                                    offset_dims=(), collapsed_slice_dims=(0,),
                                    start_index_map=(0,)),
                                slice_sizes=(1,),
                                mode=lax.GatherScatterMode.PROMISE_IN_BOUNDS)
                            g = k8 * LN + j
                            for k in range(HH // LN):
                                sl = pl.ds(k * LN, LN)
                                rows[g, sl] = rows[g, sl] * nsp
                        return 0

                    lax.fori_loop(0, 128 // LN, scale, 0)
                    pltpu.sync_copy(rows, acc.at[dv.at[b]], add=True)
                    return 0

                lax.fori_loop(0, ERT, bloop, 0)
                plsc.subcore_barrier()
                for k in range(SLR):
                    r0 = (s * SLR + k) * 128
                    pltpu.sync_copy(
                        acc.at[pl.ds(r0, 128)],
                        out.at[q, pl.ds(r0, 128), pl.ds(half * HH, HH)])
                plsc.subcore_barrier()

    return pl.kernel(
        body,
        out_type=jax.ShapeDtypeStruct((4, NACC, HQ), jnp.float32),
        mesh=plsc.VectorSubcoreMesh(**_MESH),
        scratch_types=[
            pltpu.VMEM_SHARED((NACC, HH), jnp.float32),
            pltpu.VMEM((ERT, 128), jnp.int32),
            pltpu.VMEM((ERT, 128), jnp.int32),
            pltpu.VMEM((ERT, 128), jnp.float32),
            pltpu.VMEM((128,), jnp.int32),
            pltpu.VMEM((128, HH), jnp.float32),
            pltpu.VMEM((128, HH), jnp.float32),
            pltpu.SemaphoreType.DMA,
        ],
        compiler_params=pltpu.CompilerParams(use_tc_tiling_on_sc=False),
    )


def _tc_prep_body(x_ref, w_ref, b_ref, deg_ref, h0_ref, s_ref):
    j = pl.program_id(0)
    y = jnp.dot(x_ref[...], w_ref[...],
                preferred_element_type=jnp.float32) + b_ref[...]
    hq = h0_ref.shape[-1]
    for q in range(4):
        h0_ref[q] = y[:, q * hq:(q + 1) * hq]

    @pl.when(j == 0)
    def _():
        d = deg_ref[0] + deg_ref[1]
        s_ref[...] = lax.rsqrt(jnp.maximum(d, 1e-12))


def _make_tc_prep(N, DIN, H, NACC, RB):
    HQ = H // 4
    grid = (N // RB,)
    return pl.pallas_call(
        _tc_prep_body,
        grid=grid,
        in_specs=[
            pl.BlockSpec((RB, DIN), lambda j: (j, 0)),
            pl.BlockSpec((DIN, H), lambda j: (0, 0)),
            pl.BlockSpec((1, H), lambda j: (0, 0)),
            pl.BlockSpec((NC, NACC // 128, 128), lambda j: (0, 0, 0)),
        ],
        out_specs=[
            pl.BlockSpec((4, RB, HQ), lambda j: (0, j, 0)),
            pl.BlockSpec((NACC // 128, 128), lambda j: (0, 0)),
        ],
        out_shape=[
            jax.ShapeDtypeStruct((4, N, HQ), jnp.float32),
            jax.ShapeDtypeStruct((NACC // 128, 128), jnp.float32),
        ],
    )


def _tc_layer_body(last, N, RB, agg_ref, h_ref, wl_ref, wr_ref, b_ref, g_ref,
                   be_ref, out_ref, yv, st):
    ph = pl.program_id(0)
    j = pl.program_id(1)
    hq = h_ref.shape[-1]

    @pl.when(ph == 0)
    def _():
        y = b_ref[...] + jnp.zeros((RB, 4 * hq), jnp.float32)
        for q in range(4):
            y = y + jnp.dot(agg_ref[q], wl_ref[pl.ds(q * hq, hq), :],
                            preferred_element_type=jnp.float32)
            y = y + jnp.dot(h_ref[q], wr_ref[pl.ds(q * hq, hq), :],
                            preferred_element_type=jnp.float32)
        yv[j] = y

        @pl.when(j == 0)
        def _():
            st[0:1, :] = jnp.zeros_like(st[0:1, :])
            st[1:2, :] = jnp.zeros_like(st[1:2, :])

        st[0:1, :] += jnp.sum(y, axis=0, keepdims=True)
        st[1:2, :] += jnp.sum(y * y, axis=0, keepdims=True)

    @pl.when(ph == 1)
    def _():
        @pl.when(j == 0)
        def _():
            m = st[0:1, :] * (1.0 / N)
            var = st[1:2, :] * (1.0 / N) - m * m
            sc = g_ref[...] / jnp.sqrt(var + 1e-5)
            st[2:3, :] = sc
            st[3:4, :] = be_ref[...] - m * sc

        y = yv[j]
        o = jnp.maximum(y * st[2:3, :] + st[3:4, :], 0.0)
        if last:
            res = jnp.concatenate([h_ref[q] for q in range(4)], axis=1)
            out_ref[...] = o + res
        else:
            for q in range(4):
                out_ref[q] = o[:, q * hq:(q + 1) * hq] + h_ref[q]


def _make_tc_layer(N, H, NACC, RB, last):
    HQ = H // 4
    grid = (2, N // RB)
    if last:
        out_spec = pl.BlockSpec((RB, H), lambda ph, j: (j, 0))
        out_shape = jax.ShapeDtypeStruct((N, H), jnp.float32)
    else:
        out_spec = pl.BlockSpec((4, RB, HQ), lambda ph, j: (0, j, 0))
        out_shape = jax.ShapeDtypeStruct((4, N, HQ), jnp.float32)
    return pl.pallas_call(
        functools.partial(_tc_layer_body, last, N, RB),
        grid=grid,
        in_specs=[
            pl.BlockSpec((4, RB, HQ), lambda ph, j: (0, j, 0)),
            pl.BlockSpec((4, RB, HQ), lambda ph, j: (0, j, 0)),
            pl.BlockSpec((H, H), lambda ph, j: (0, 0)),
            pl.BlockSpec((H, H), lambda ph, j: (0, 0)),
            pl.BlockSpec((1, H), lambda ph, j: (0, 0)),
            pl.BlockSpec((1, H), lambda ph, j: (0, 0)),
            pl.BlockSpec((1, H), lambda ph, j: (0, 0)),
        ],
        out_specs=out_spec,
        out_shape=out_shape,
        scratch_shapes=[
            pltpu.VMEM((N // RB, RB, H), jnp.float32),
            pltpu.VMEM((8, H), jnp.float32),
        ],
        compiler_params=pltpu.CompilerParams(
            dimension_semantics=("arbitrary", "arbitrary")),
    )


def kernel(x, edge_index, edge_attr, W_in, b_in, W_rel, b_rel, W_root, gamma,
           beta):
    N, DIN = x.shape
    H = W_in.shape[1]
    L = W_rel.shape[0]
    E = edge_index.shape[1]
    HQ = H // 4
    RB = 1000

    EPAD = _round_up(E, NC * NS * 128)
    NACC = _round_up(N + 1, NS * 128)

    src = edge_index[0]
    dst = edge_index[1]
    w = edge_attr.reshape(-1)
    pad = EPAD - E
    srcp = jnp.concatenate(
        [src, jnp.zeros((pad,), jnp.int32)]).reshape(-1, 128)
    dstp = jnp.concatenate(
        [dst, jnp.full((pad,), N, jnp.int32)]).reshape(-1, 128)
    wp = jnp.concatenate(
        [w, jnp.zeros((pad,), jnp.float32)]).reshape(-1, 128)

    degp = _make_deg(EPAD, NACC)(dstp, wp)
    h0q, s_tab = _make_tc_prep(N, DIN, H, NACC, RB)(
        x, W_in, b_in[None], degp.reshape(NC, NACC // 128, 128))
    normp = _make_norm(EPAD, NACC)(srcp, dstp, wp, s_tab.reshape(-1))

    agg_fn = _make_agg(EPAD, NACC, N, HQ)
    h = h0q
    for i in range(L):
        aggq = agg_fn(h.reshape(8 * N, HQ // 2), srcp, dstp, normp)
        layer_fn = _make_tc_layer(N, H, NACC, RB, last=(i == L - 1))
        h = layer_fn(aggq, h, W_rel[i], W_root[i], b_rel[i][None],
                     gamma[i][None], beta[i][None])
    return h


# speedup vs baseline: 1.9944x; 1.1403x over previous
import functools

import jax
import jax.numpy as jnp
from jax import lax
from jax.experimental import pallas as pl
from jax.experimental.pallas import tpu as pltpu
from jax.experimental.pallas import tpu_sc as plsc

NC = 2
NS = 16
LN = 16

_MESH = dict(core_axis_name="c", subcore_axis_name="s", num_cores=NC,
             num_subcores=NS)


def _round_up(a, b):
    return (a + b - 1) // b * b


def _make_deg(EPAD, NACC):
    ER = EPAD // 128
    ERW = ER // (NC * NS)
    SLC = NACC // NS

    def body(dstp, wp, out, acc, dv, wv, zb):
        c = lax.axis_index("c")
        s = lax.axis_index("s")
        row0 = (c * NS + s) * ERW
        pltpu.sync_copy(dstp.at[pl.ds(row0, ERW)], dv)
        pltpu.sync_copy(wp.at[pl.ds(row0, ERW)], wv)
        z = jnp.zeros((LN,), jnp.float32)

        def zloop(i, _):
            zb[pl.ds(i * LN, LN)] = z
            return 0

        lax.fori_loop(0, SLC // LN, zloop, 0)
        pltpu.sync_copy(zb, acc.at[pl.ds(s * SLC, SLC)])
        plsc.subcore_barrier()

        def jloop(j, _):
            pltpu.sync_copy(wv.at[j], acc.at[dv.at[j]], add=True)
            return 0

        lax.fori_loop(0, ERW, jloop, 0)
        plsc.subcore_barrier()
        pltpu.sync_copy(acc.at[pl.ds(s * SLC, SLC)],
                        out.at[c, pl.ds(s * SLC, SLC)])

    return pl.kernel(
        body,
        out_type=jax.ShapeDtypeStruct((NC, NACC), jnp.float32),
        mesh=plsc.VectorSubcoreMesh(**_MESH),
        scratch_types=[
            pltpu.VMEM_SHARED((NACC,), jnp.float32),
            pltpu.VMEM((ERW, 128), jnp.int32),
            pltpu.VMEM((ERW, 128), jnp.float32),
            pltpu.VMEM((SLC,), jnp.float32),
        ],
    )


def _make_norm(EPAD, NACC):
    ER = EPAD // 128
    ERW = ER // (NC * NS)
    SR = NACC // 128

    def body(srcp, dstp, wp, s_hbm, out, sv, dv, wv, ov, ssb, sdb, sem):
        c = lax.axis_index("c")
        s = lax.axis_index("s")
        row0 = (c * NS + s) * ERW
        pltpu.sync_copy(srcp.at[pl.ds(row0, ERW)], sv)
        pltpu.sync_copy(dstp.at[pl.ds(row0, ERW)], dv)
        pltpu.sync_copy(wp.at[pl.ds(row0, ERW)], wv)

        def rloop(r, _):
            d1 = pltpu.async_copy(s_hbm.at[sv.at[r]], ssb, sem)
            d2 = pltpu.async_copy(s_hbm.at[dv.at[r]], sdb, sem)
            d1.wait()
            d2.wait()
            for k in range(128 // LN):
                sl = pl.ds(k * LN, LN)
                ov[r, sl] = wv[r, sl] * ssb[sl] * sdb[sl]
            return 0

        lax.fori_loop(0, ERW, rloop, 0)
        pltpu.sync_copy(ov, out.at[pl.ds(row0, ERW)])

    return pl.kernel(
        body,
        out_type=jax.ShapeDtypeStruct((ER, 128), jnp.float32),
        mesh=plsc.VectorSubcoreMesh(**_MESH),
        scratch_types=[
            pltpu.VMEM((ERW, 128), jnp.int32),
            pltpu.VMEM((ERW, 128), jnp.int32),
            pltpu.VMEM((ERW, 128), jnp.float32),
            pltpu.VMEM((ERW, 128), jnp.float32),
            pltpu.VMEM((128,), jnp.float32),
            pltpu.VMEM((128,), jnp.float32),
            pltpu.SemaphoreType.DMA,
        ],
    )


def _make_agg(EPAD, NACC, N, HQ):
    ER = EPAD // 128
    ERT = ER // NS
    SLR = NACC // NS // 128
    HH = HQ // 2

    def body(htab, srcp, dstp, normp, out, acc, sv, dv, nv, gidx0, gidx1,
             rows0, rows1, zb, sg0, sg1, ss0, ss1):
        c = lax.axis_index("c")
        s = lax.axis_index("s")
        row0 = s * ERT
        pltpu.sync_copy(srcp.at[pl.ds(row0, ERT)], sv)
        pltpu.sync_copy(dstp.at[pl.ds(row0, ERT)], dv)
        pltpu.sync_copy(normp.at[pl.ds(row0, ERT)], nv)
        z = jnp.zeros((LN,), jnp.float32)

        def zloop(i, _):
            for k in range(HH // LN):
                zb[i, pl.ds(k * LN, LN)] = z
            return 0

        lax.fori_loop(0, 128, zloop, 0)

        def fill_gidx(gidx, b, tb):
            for k in range(128 // LN):
                sl = pl.ds(k * LN, LN)
                gidx[sl] = sv[b, sl] * 2 + tb

        def scale(rows, b):
            def sc8(k8, _):
                n16 = nv[b, pl.ds(k8 * LN, LN)]
                for j in range(LN):
                    nsp = lax.gather(
                        n16, jnp.full((LN, 1), j, jnp.int32),
                        lax.GatherDimensionNumbers(
                            offset_dims=(), collapsed_slice_dims=(0,),
                            start_index_map=(0,)),
                        slice_sizes=(1,),
                        mode=lax.GatherScatterMode.PROMISE_IN_BOUNDS)
                    g = k8 * LN + j
                    for k in range(HH // LN):
                        sl = pl.ds(k * LN, LN)
                        rows[g, sl] = rows[g, sl] * nsp
                return 0

            lax.fori_loop(0, 128 // LN, sc8, 0)

        for qi in range(4 // NC):
            q = c * (4 // NC) + qi
            for half in range(2):
                tb = q * 2 * N + half
                for k in range(SLR):
                    pltpu.sync_copy(
                        zb, acc.at[pl.ds((s * SLR + k) * 128, 128)])
                plsc.subcore_barrier()

                def bloop(b2, _):
                    b = 2 * b2
                    fill_gidx(gidx0, b, tb)
                    d0 = pltpu.async_copy(htab.at[gidx0], rows0, sg0)
                    fill_gidx(gidx1, b + 1, tb)
                    d1 = pltpu.async_copy(htab.at[gidx1], rows1, sg1)
                    d0.wait()
                    scale(rows0, b)
                    s0 = pltpu.async_copy(rows0, acc.at[dv.at[b]], ss0,
                                          add=True)
                    d1.wait()
                    scale(rows1, b + 1)
                    s1 = pltpu.async_copy(rows1, acc.at[dv.at[b + 1]], ss1,
                                          add=True)
                    s0.wait()
                    s1.wait()
                    return 0

                lax.fori_loop(0, ERT // 2, bloop, 0)
                plsc.subcore_barrier()
                for k in range(SLR):
                    r0 = (s * SLR + k) * 128
                    pltpu.sync_copy(
                        acc.at[pl.ds(r0, 128)],
                        out.at[q, pl.ds(r0, 128), pl.ds(half * HH, HH)])
                plsc.subcore_barrier()

    return pl.kernel(
        body,
        out_type=jax.ShapeDtypeStruct((4, NACC, HQ), jnp.float32),
        mesh=plsc.VectorSubcoreMesh(**_MESH),
        scratch_types=[
            pltpu.VMEM_SHARED((NACC, HH), jnp.float32),
            pltpu.VMEM((ERT, 128), jnp.int32),
            pltpu.VMEM((ERT, 128), jnp.int32),
            pltpu.VMEM((ERT, 128), jnp.float32),
            pltpu.VMEM((128,), jnp.int32),
            pltpu.VMEM((128,), jnp.int32),
            pltpu.VMEM((128, HH), jnp.float32),
            pltpu.VMEM((128, HH), jnp.float32),
            pltpu.VMEM((128, HH), jnp.float32),
            pltpu.SemaphoreType.DMA,
            pltpu.SemaphoreType.DMA,
            pltpu.SemaphoreType.DMA,
            pltpu.SemaphoreType.DMA,
        ],
        compiler_params=pltpu.CompilerParams(use_tc_tiling_on_sc=False),
    )


def _tc_prep_body(x_ref, w_ref, b_ref, deg_ref, h0_ref, s_ref):
    j = pl.program_id(0)
    y = jnp.dot(x_ref[...], w_ref[...],
                preferred_element_type=jnp.float32) + b_ref[...]
    hq = h0_ref.shape[-1]
    for q in range(4):
        h0_ref[q] = y[:, q * hq:(q + 1) * hq]

    @pl.when(j == 0)
    def _():
        d = deg_ref[0] + deg_ref[1]
        s_ref[...] = lax.rsqrt(jnp.maximum(d, 1e-12))


def _make_tc_prep(N, DIN, H, NACC, RB):
    HQ = H // 4
    grid = (N // RB,)
    return pl.pallas_call(
        _tc_prep_body,
        grid=grid,
        in_specs=[
            pl.BlockSpec((RB, DIN), lambda j: (j, 0)),
            pl.BlockSpec((DIN, H), lambda j: (0, 0)),
            pl.BlockSpec((1, H), lambda j: (0, 0)),
            pl.BlockSpec((NC, NACC // 128, 128), lambda j: (0, 0, 0)),
        ],
        out_specs=[
            pl.BlockSpec((4, RB, HQ), lambda j: (0, j, 0)),
            pl.BlockSpec((NACC // 128, 128), lambda j: (0, 0)),
        ],
        out_shape=[
            jax.ShapeDtypeStruct((4, N, HQ), jnp.float32),
            jax.ShapeDtypeStruct((NACC // 128, 128), jnp.float32),
        ],
    )


def _tc_layer_body(last, N, RB, agg_ref, h_ref, wl_ref, wr_ref, b_ref, g_ref,
                   be_ref, out_ref, yv, st):
    ph = pl.program_id(0)
    j = pl.program_id(1)
    hq = h_ref.shape[-1]

    @pl.when(ph == 0)
    def _():
        y = b_ref[...] + jnp.zeros((RB, 4 * hq), jnp.float32)
        for q in range(4):
            y = y + jnp.dot(agg_ref[q], wl_ref[pl.ds(q * hq, hq), :],
                            preferred_element_type=jnp.float32)
            y = y + jnp.dot(h_ref[q], wr_ref[pl.ds(q * hq, hq), :],
                            preferred_element_type=jnp.float32)
        yv[j] = y

        @pl.when(j == 0)
        def _():
            st[0:1, :] = jnp.zeros_like(st[0:1, :])
            st[1:2, :] = jnp.zeros_like(st[1:2, :])

        st[0:1, :] += jnp.sum(y, axis=0, keepdims=True)
        st[1:2, :] += jnp.sum(y * y, axis=0, keepdims=True)

    @pl.when(ph == 1)
    def _():
        @pl.when(j == 0)
        def _():
            m = st[0:1, :] * (1.0 / N)
            var = st[1:2, :] * (1.0 / N) - m * m
            sc = g_ref[...] / jnp.sqrt(var + 1e-5)
            st[2:3, :] = sc
            st[3:4, :] = be_ref[...] - m * sc

        y = yv[j]
        o = jnp.maximum(y * st[2:3, :] + st[3:4, :], 0.0)
        if last:
            res = jnp.concatenate([h_ref[q] for q in range(4)], axis=1)
            out_ref[...] = o + res
        else:
            for q in range(4):
                out_ref[q] = o[:, q * hq:(q + 1) * hq] + h_ref[q]


def _make_tc_layer(N, H, NACC, RB, last):
    HQ = H // 4
    grid = (2, N // RB)
    if last:
        out_spec = pl.BlockSpec((RB, H), lambda ph, j: (j, 0))
        out_shape = jax.ShapeDtypeStruct((N, H), jnp.float32)
    else:
        out_spec = pl.BlockSpec((4, RB, HQ), lambda ph, j: (0, j, 0))
        out_shape = jax.ShapeDtypeStruct((4, N, HQ), jnp.float32)
    return pl.pallas_call(
        functools.partial(_tc_layer_body, last, N, RB),
        grid=grid,
        in_specs=[
            pl.BlockSpec((4, RB, HQ), lambda ph, j: (0, j, 0)),
            pl.BlockSpec((4, RB, HQ), lambda ph, j: (0, j, 0)),
            pl.BlockSpec((H, H), lambda ph, j: (0, 0)),
            pl.BlockSpec((H, H), lambda ph, j: (0, 0)),
            pl.BlockSpec((1, H), lambda ph, j: (0, 0)),
            pl.BlockSpec((1, H), lambda ph, j: (0, 0)),
            pl.BlockSpec((1, H), lambda ph, j: (0, 0)),
        ],
        out_specs=out_spec,
        out_shape=out_shape,
        scratch_shapes=[
            pltpu.VMEM((N // RB, RB, H), jnp.float32),
            pltpu.VMEM((8, H), jnp.float32),
        ],
        compiler_params=pltpu.CompilerParams(
            dimension_semantics=("arbitrary", "arbitrary")),
    )


def kernel(x, edge_index, edge_attr, W_in, b_in, W_rel, b_rel, W_root, gamma,
           beta):
    N, DIN = x.shape
    H = W_in.shape[1]
    L = W_rel.shape[0]
    E = edge_index.shape[1]
    HQ = H // 4
    RB = 1000

    EPAD = _round_up(E, NC * NS * 128)
    NACC = _round_up(N + 1, NS * 128)

    src = edge_index[0]
    dst = edge_index[1]
    w = edge_attr.reshape(-1)
    pad = EPAD - E
    srcp = jnp.concatenate(
        [src, jnp.zeros((pad,), jnp.int32)]).reshape(-1, 128)
    dstp = jnp.concatenate(
        [dst, jnp.full((pad,), N, jnp.int32)]).reshape(-1, 128)
    wp = jnp.concatenate(
        [w, jnp.zeros((pad,), jnp.float32)]).reshape(-1, 128)

    degp = _make_deg(EPAD, NACC)(dstp, wp)
    h0q, s_tab = _make_tc_prep(N, DIN, H, NACC, RB)(
        x, W_in, b_in[None], degp.reshape(NC, NACC // 128, 128))
    normp = _make_norm(EPAD, NACC)(srcp, dstp, wp, s_tab.reshape(-1))

    agg_fn = _make_agg(EPAD, NACC, N, HQ)
    h = h0q
    for i in range(L):
        aggq = agg_fn(h.reshape(8 * N, HQ // 2), srcp, dstp, normp)
        layer_fn = _make_tc_layer(N, H, NACC, RB, last=(i == L - 1))
        h = layer_fn(aggq, h, W_rel[i], W_root[i], b_rel[i][None],
                     gamma[i][None], beta[i][None])
    return h
